# Initial kernel scaffold; baseline (speedup 1.0000x reference)
import jax
import jax.numpy as jnp
from jax.experimental import pallas as pl


def kernel(x, seg, tok_emb, pos_emb, seg_emb, gamma, beta):
    raise NotImplementedError("write your pallas kernel here")


# speedup vs baseline: 3.0514x; 3.0514x over previous
import functools

import jax
import jax.numpy as jnp
from jax import lax
from jax.experimental import pallas as pl
from jax.experimental.pallas import tpu as pltpu
from jax.experimental.pallas import tpu_sc as plsc

_NW = 32
_CH = 128


@functools.lru_cache(maxsize=None)
def _make_sc_gather(V, D, N):
    assert N % (_NW * _CH) == 0
    nch = N // (_NW * _CH)
    mesh = plsc.VectorSubcoreMesh(core_axis_name="c", subcore_axis_name="s")

    @functools.partial(
        pl.kernel,
        mesh=mesh,
        out_type=jax.ShapeDtypeStruct((N, D), jnp.float32),
        scratch_types=[
            pltpu.VMEM((nch, _CH), jnp.int32),
            pltpu.VMEM((_CH, D), jnp.float32),
            pltpu.SemaphoreType.DMA,
        ],
    )
    def sc_gather(table_hbm, idx_hbm, out_hbm, idx_v, rows_v, sem):
        wid = lax.axis_index("s") * 2 + lax.axis_index("c")
        pltpu.sync_copy(idx_hbm.at[wid], idx_v)
        base = wid * (nch * _CH)

        def body(ch, carry):
            pltpu.async_copy(table_hbm.at[idx_v.at[ch]], rows_v, sem).wait()
            pltpu.sync_copy(rows_v, out_hbm.at[pl.ds(base + ch * _CH, _CH)])
            return carry

        lax.fori_loop(0, nch, body, 0)

    return sc_gather


@functools.lru_cache(maxsize=None)
def _make_tc_fuse_ln(B, L, D, bb=8):
    assert B % bb == 0

    def body(emb_ref, seg_ref, pos_ref, se_ref, g_ref, b_ref, out_ref):
        h = emb_ref[...]
        s = seg_ref[...].astype(jnp.float32)[:, :, None]
        e0 = se_ref[0, :][None, None, :]
        e1 = se_ref[1, :][None, None, :]
        h = h + pos_ref[...][None, :, :] + e0 + s * (e1 - e0)
        mean = jnp.mean(h, axis=-1, keepdims=True)
        c = h - mean
        var = jnp.mean(c * c, axis=-1, keepdims=True)
        out_ref[...] = c * lax.rsqrt(var + 1e-5) * g_ref[...] + b_ref[...]

    return pl.pallas_call(
        body,
        grid=(B // bb,),
        in_specs=[
            pl.BlockSpec((bb, L, D), lambda i: (i, 0, 0)),
            pl.BlockSpec((bb, L), lambda i: (i, 0)),
            pl.BlockSpec((L, D), lambda i: (0, 0)),
            pl.BlockSpec((2, D), lambda i: (0, 0)),
            pl.BlockSpec((D,), lambda i: (0,)),
            pl.BlockSpec((D,), lambda i: (0,)),
        ],
        out_specs=pl.BlockSpec((bb, L, D), lambda i: (i, 0, 0)),
        out_shape=jax.ShapeDtypeStruct((B, L, D), jnp.float32),
    )


def kernel(x, seg, tok_emb, pos_emb, seg_emb, gamma, beta):
    B, L = x.shape
    V, D = tok_emb.shape
    N = B * L
    emb = jnp.take(tok_emb, x, axis=0)
    emb = emb.reshape(B, L, D)
    return _make_tc_fuse_ln(B, L, D)(
        emb, seg.astype(jnp.int32), pos_emb[:L], seg_emb, gamma, beta
    )
